# Initial kernel scaffold; baseline (speedup 1.0000x reference)
import jax
import jax.numpy as jnp
from jax.experimental import pallas as pl


def kernel(x, gmf_user, gmf_movie, mlp_user, mlp_movie, W1, b1, W2, b2, W3, b3):
    raise NotImplementedError("write your pallas kernel here")


# speedup vs baseline: 2.2725x; 2.2725x over previous
import functools

import jax
import jax.numpy as jnp
from jax import lax
from jax.experimental import pallas as pl
from jax.experimental.pallas import tpu as pltpu
from jax.experimental.pallas import tpu_sc as plsc

B = 16384
D = 128
H = 512

NC = 2
NS = 16
NW = NC * NS
BPW = B // NW


def _make_sc_gather(V1):
  mesh = plsc.VectorSubcoreMesh(core_axis_name="c", subcore_axis_name="s")

  @functools.partial(
      pl.kernel,
      mesh=mesh,
      out_type=[jax.ShapeDtypeStruct((B, D), jnp.float32) for _ in range(4)],
      scratch_types=[
          pltpu.VMEM((BPW,), jnp.int32),
          pltpu.VMEM((BPW,), jnp.int32),
          pltpu.VMEM((BPW, D), jnp.float32),
          pltpu.SemaphoreType.DMA,
      ],
  )
  def sc_gather(uidx_hbm, midx_hbm, gu_hbm, gm_hbm, mu_hbm, mm_hbm,
                eu_out, em_out, muo_out, mmo_out,
                uidx_v, midx_v, buf, sem):
    wid = lax.axis_index("s") * NC + lax.axis_index("c")
    base = wid * BPW
    pltpu.sync_copy(uidx_hbm.at[pl.ds(base, BPW)], uidx_v)
    pltpu.sync_copy(midx_hbm.at[pl.ds(base, BPW)], midx_v)
    for table, idx_v, out in ((gu_hbm, uidx_v, eu_out),
                              (gm_hbm, midx_v, em_out),
                              (mu_hbm, uidx_v, muo_out),
                              (mm_hbm, midx_v, mmo_out)):
      pltpu.async_copy(table.at[idx_v], buf, sem).wait()
      pltpu.sync_copy(buf, out.at[pl.ds(base, BPW)])

  return sc_gather


def _tc_dense_body(eu, em, mu, mm, W1r, b1r, W2r, b2r, w3ur, w3mr, b3r,
                   out, wc, bc):
  i = pl.program_id(0)

  @pl.when(i == 0)
  def _():
    wc[...] = jnp.dot(W1r[...], W2r[...], preferred_element_type=jnp.float32)
    bc[...] = (jnp.dot(b1r[...], W2r[...], preferred_element_type=jnp.float32)
               + b2r[...])

  h = (jnp.dot(mu[...], wc[0:D, :], preferred_element_type=jnp.float32)
       + jnp.dot(mm[...], wc[D:2 * D, :], preferred_element_type=jnp.float32)
       + bc[...])
  hr = jnp.maximum(h, 0.0)
  g = eu[...] * em[...]
  o = (jnp.sum(g * w3ur[...], axis=1, keepdims=True)
       + jnp.sum(hr * w3mr[...], axis=1, keepdims=True)
       + b3r[...])
  out[...] = o


def _tc_dense(eu, em, mu, mm, W1, b1, W2, b2, W3, b3):
  bs = 512
  grid = (B // bs,)
  row = lambda i: (i, 0)
  const = lambda i: (0, 0)
  return pl.pallas_call(
      _tc_dense_body,
      grid=grid,
      in_specs=[
          pl.BlockSpec((bs, D), row),
          pl.BlockSpec((bs, D), row),
          pl.BlockSpec((bs, D), row),
          pl.BlockSpec((bs, D), row),
          pl.BlockSpec((2 * D, H), const),
          pl.BlockSpec((1, H), const),
          pl.BlockSpec((H, 2 * D), const),
          pl.BlockSpec((1, 2 * D), const),
          pl.BlockSpec((1, D), const),
          pl.BlockSpec((1, 2 * D), const),
          pl.BlockSpec((1, 1), const),
      ],
      out_specs=pl.BlockSpec((bs, 1), row),
      out_shape=jax.ShapeDtypeStruct((B, 1), jnp.float32),
      scratch_shapes=[
          pltpu.VMEM((2 * D, 2 * D), jnp.float32),
          pltpu.VMEM((1, 2 * D), jnp.float32),
      ],
      compiler_params=pltpu.CompilerParams(
          dimension_semantics=("arbitrary",)),
  )(eu, em, mu, mm, W1, b1.reshape(1, H), W2, b2.reshape(1, 2 * D),
    W3[:D, 0].reshape(1, D), W3[D:, 0].reshape(1, 2 * D),
    b3.reshape(1, 1))


def kernel(x, gmf_user, gmf_movie, mlp_user, mlp_movie, W1, b1, W2, b2, W3,
           b3):
  user = x[:, 0]
  movie = x[:, 1]
  rating = x[:, 2]
  sc_gather = _make_sc_gather(gmf_user.shape[0])
  eu, em, mu, mm = sc_gather(user, movie, gmf_user, gmf_movie, mlp_user,
                             mlp_movie)
  out = _tc_dense(eu, em, mu, mm, W1, b1, W2, b2, W3, b3)
  return out, rating
